# Initial kernel scaffold; baseline (speedup 1.0000x reference)
import jax
import jax.numpy as jnp
from jax.experimental import pallas as pl


def kernel(z):
    raise NotImplementedError("write your pallas kernel here")


# speedup vs baseline: 1.8615x; 1.8615x over previous
import functools

import jax
import jax.numpy as jnp
from jax import lax
from jax.experimental import pallas as pl
from jax.experimental.pallas import tpu as pltpu
from jax.experimental.pallas import tpu_sc as plsc

N = 8192
D = 4096
L = 16
NC = 2
NS = 16
NW = NC * NS
ROWS_PER_W = N // NW
CHUNK = 8
VECS = D // L
NCHUNKS = ROWS_PER_W // CHUNK

_mesh = plsc.VectorSubcoreMesh(core_axis_name="c", subcore_axis_name="s")


@functools.partial(
    pl.kernel,
    out_type=jax.ShapeDtypeStruct((N, D), jnp.float32),
    mesh=_mesh,
    scratch_types=[
        pltpu.VMEM((CHUNK, D), jnp.float32),
        pltpu.VMEM((CHUNK, D), jnp.float32),
    ],
)
def _reverse_rows(z_hbm, out_hbm, in_v, out_v):
    wid = lax.axis_index("s") * NC + lax.axis_index("c")
    base = wid * ROWS_PER_W

    def chunk_body(ci, _):
        row0 = base + ci * CHUNK
        pltpu.sync_copy(z_hbm.at[pl.ds(row0, CHUNK)], in_v)

        @plsc.parallel_loop(0, VECS, unroll=8)
        def _(k):
            src = k * L
            dst = (VECS - 1 - k) * L
            for r in range(CHUNK):
                v = in_v[r, pl.ds(src, L)]
                out_v[r, pl.ds(dst, L)] = lax.rev(v, dimensions=(0,))

        pltpu.sync_copy(out_v, out_hbm.at[pl.ds(row0, CHUNK)])
        return 0

    lax.fori_loop(0, NCHUNKS, chunk_body, 0)


def kernel(z):
    out = _reverse_rows(z)
    log_det = jnp.zeros((N, 1), dtype=z.dtype)
    return (out, log_det)


# speedup vs baseline: 2.7358x; 1.4697x over previous
import functools

import jax
import jax.numpy as jnp
from jax import lax
from jax.experimental import pallas as pl
from jax.experimental.pallas import tpu as pltpu
from jax.experimental.pallas import tpu_sc as plsc

N = 8192
D = 4096
L = 16
NC = 2
NS = 16
NW = NC * NS
ROWS_PER_W = N // NW
CHUNK = 4
VECS = D // L
NCHUNKS = ROWS_PER_W // CHUNK
BUFS = 4
NGROUPS = NCHUNKS // BUFS

_mesh = plsc.VectorSubcoreMesh(core_axis_name="c", subcore_axis_name="s")


@functools.partial(
    pl.kernel,
    out_type=jax.ShapeDtypeStruct((N, D), jnp.float32),
    mesh=_mesh,
    scratch_types=(
        [pltpu.VMEM((CHUNK, D), jnp.float32)] * BUFS
        + [pltpu.SemaphoreType.DMA] * (2 * BUFS)
    ),
)
def _reverse_rows(z_hbm, out_hbm, b0, b1, b2, b3, si0, si1, si2, si3,
                  so0, so1, so2, so3):
    bufs = (b0, b1, b2, b3)
    sin = (si0, si1, si2, si3)
    sout = (so0, so1, so2, so3)

    wid = lax.axis_index("s") * NC + lax.axis_index("c")
    base = wid * ROWS_PER_W

    def rows_of(ci):
        return pl.ds(base + ci * CHUNK, CHUNK)

    def issue_in(ci, b):
        pltpu.async_copy(z_hbm.at[rows_of(ci)], bufs[b], sin[b])

    def wait_in(ci, b):
        pltpu.make_async_copy(z_hbm.at[rows_of(ci)], bufs[b], sin[b]).wait()

    def issue_out(ci, b):
        pltpu.async_copy(bufs[b], out_hbm.at[rows_of(ci)], sout[b])

    def wait_out(ci, b):
        pltpu.make_async_copy(bufs[b], out_hbm.at[rows_of(ci)], sout[b]).wait()

    def compute(b):
        buf = bufs[b]

        @plsc.parallel_loop(0, VECS // 2, unroll=4)
        def _(k):
            lo = k * L
            hi = (VECS - 1 - k) * L
            for r in range(CHUNK):
                va = buf[r, pl.ds(lo, L)]
                vb = buf[r, pl.ds(hi, L)]
                buf[r, pl.ds(lo, L)] = lax.rev(vb, dimensions=(0,))
                buf[r, pl.ds(hi, L)] = lax.rev(va, dimensions=(0,))

    def step(ci, b, head, tail):
        if not head:
            wait_out(ci - 2, (b + 2) % BUFS)
        if not tail:
            issue_in(ci + 2, (b + 2) % BUFS)
        wait_in(ci, b)
        compute(b)
        issue_out(ci, b)

    issue_in(0, 0)
    issue_in(1, 1)

    for b in range(BUFS):
        step(b, b, head=(b < 2), tail=False)

    def group_body(g, _):
        for b in range(BUFS):
            step(g * BUFS + b, b, head=False, tail=False)
        return 0

    lax.fori_loop(1, NGROUPS - 1, group_body, 0)

    for b in range(BUFS):
        ci = (NGROUPS - 1) * BUFS + b
        step(ci, b, head=False, tail=(b >= 2))

    wait_out(NCHUNKS - 2, (NCHUNKS - 2) % BUFS)
    wait_out(NCHUNKS - 1, (NCHUNKS - 1) % BUFS)


def kernel(z):
    out = _reverse_rows(z)
    log_det = jnp.zeros((N, 1), dtype=z.dtype)
    return (out, log_det)
